# Initial kernel scaffold; baseline (speedup 1.0000x reference)
import jax
import jax.numpy as jnp
from jax.experimental import pallas as pl


def kernel(new_ft, raw_ft, edge_index):
    raise NotImplementedError("write your pallas kernel here")


# speedup vs baseline: 2.4780x; 2.4780x over previous
import functools

import jax
import jax.numpy as jnp
from jax import lax
from jax.experimental import pallas as pl
from jax.experimental.pallas import tpu as pltpu
from jax.experimental.pallas import tpu_sc as plsc

N_NODES = 10000
N_EDGES = 160000
D_FEAT = 256

NC = 2
NS = 16
NW = NC * NS
PER_W = N_EDGES // NW
CHUNK = 200
NCHUNK = PER_W // CHUNK
LANES = 16
NGROUP = (CHUNK + LANES - 1) // LANES
CHUNK_PAD = NGROUP * LANES

_BITREV = [int("{:04b}".format(i)[::-1], 2) for i in range(LANES)]


def _rot(v, s, idx):
    dnums = lax.GatherDimensionNumbers(
        offset_dims=(), collapsed_slice_dims=(0,), start_index_map=(0,))
    return lax.gather(v, (idx ^ s)[:, None], dnums, (1,),
                      mode=lax.GatherScatterMode.PROMISE_IN_BOUNDS)


def _tree16(vs, idx):
    level = vs
    for s in (8, 4, 2, 1):
        nxt = []
        for i in range(0, len(level), 2):
            a, b = level[i], level[i + 1]
            nxt.append(jnp.where((idx & (2 * s - 1)) < s,
                                 a + _rot(a, s, idx), b + _rot(b, s, idx)))
        level = nxt
    return level[0]


def _make_sc_kernel():
    mesh = plsc.VectorSubcoreMesh(core_axis_name="c", subcore_axis_name="s")

    @functools.partial(
        pl.kernel,
        mesh=mesh,
        out_type=jax.ShapeDtypeStruct((N_EDGES,), jnp.float32),
        scratch_types=[
            pltpu.VMEM((PER_W,), jnp.int32),
            pltpu.VMEM((PER_W,), jnp.int32),
            pltpu.VMEM((CHUNK, D_FEAT), jnp.float32),
            pltpu.VMEM((CHUNK, D_FEAT), jnp.float32),
            pltpu.VMEM((CHUNK_PAD,), jnp.float32),
            pltpu.SemaphoreType.DMA,
            pltpu.SemaphoreType.DMA,
        ],
    )
    def k(new_hbm, raw_hbm, src_hbm, dst_hbm, out_hbm,
          src_v, dst_v, u_rows, v_rows, out_v, sem_u, sem_v):
        wid = lax.axis_index("s") * NC + lax.axis_index("c")
        base = wid * PER_W
        pltpu.sync_copy(src_hbm.at[pl.ds(base, PER_W)], src_v)
        pltpu.sync_copy(dst_hbm.at[pl.ds(base, PER_W)], dst_v)

        idx = lax.iota(jnp.int32, LANES)

        def chunk_body(j, _):
            off = j * CHUNK
            cu = pltpu.async_copy(
                new_hbm.at[src_v.at[pl.ds(off, CHUNK)]], u_rows, sem_u)
            cv = pltpu.async_copy(
                raw_hbm.at[dst_v.at[pl.ds(off, CHUNK)]], v_rows, sem_v)
            cu.wait()
            cv.wait()

            def group_body(g, _):
                e0 = g * LANES
                accs = []
                for p in range(LANES):
                    l = _BITREV[p]
                    e = e0 + l
                    e = jnp.where(e < CHUNK, e, 0)
                    acc = (u_rows[e, pl.ds(0, LANES)]
                           * v_rows[e, pl.ds(0, LANES)])
                    for kk in range(1, D_FEAT // LANES):
                        acc += (u_rows[e, pl.ds(kk * LANES, LANES)]
                                * v_rows[e, pl.ds(kk * LANES, LANES)])
                    accs.append(acc)
                out_v[pl.ds(e0, LANES)] = _tree16(accs, idx)
                return 0

            lax.fori_loop(0, NGROUP, group_body, 0)
            pltpu.sync_copy(out_v.at[pl.ds(0, CHUNK)],
                            out_hbm.at[pl.ds(base + off, CHUNK)])
            return 0

        lax.fori_loop(0, NCHUNK, chunk_body, 0)

    return k


_sc_kernel = _make_sc_kernel()


@jax.jit
def kernel(new_ft, raw_ft, edge_index):
    src = edge_index[0].astype(jnp.int32)
    dst = edge_index[1].astype(jnp.int32)
    score = _sc_kernel(new_ft, raw_ft, src, dst)
    return score.reshape(N_EDGES, 1)
